# Initial kernel scaffold; baseline (speedup 1.0000x reference)
import jax
import jax.numpy as jnp
from jax.experimental import pallas as pl


def kernel(x, weight, bias):
    raise NotImplementedError("write your pallas kernel here")


# speedup vs baseline: 55.1728x; 55.1728x over previous
import jax
import jax.numpy as jnp
from jax.experimental import pallas as pl
from jax.experimental.pallas import tpu as pltpu

_CIN = 4
_COUT = 8


def _conv3x3_body(w_ref, x_ref, o_ref):
    xt = x_ref[0]
    cin, h, w = xt.shape

    zrow = jnp.zeros((cin, 1, w), jnp.float32)
    rows = [
        jnp.concatenate([zrow, xt[:, :-1, :]], axis=1),
        xt,
        jnp.concatenate([xt[:, 1:, :], zrow], axis=1),
    ]
    zcol = jnp.zeros((cin, h, 1), jnp.float32)
    taps = []
    for r in rows:
        taps.append(jnp.concatenate([zcol, r[:, :, :-1]], axis=2))
        taps.append(r)
        taps.append(jnp.concatenate([r[:, :, 1:], zcol], axis=2))

    for co in range(_COUT):
        acc = jnp.full((h, w), w_ref[co, _CIN * 9], jnp.float32)
        for ci in range(cin):
            for t in range(9):
                acc = acc + w_ref[co, ci * 9 + t] * taps[t][ci]
        o_ref[0, co] = acc


def kernel(x, weight, bias):
    B, Cin, H, W = x.shape
    Cout = weight.shape[0]
    w2 = jnp.concatenate(
        [weight.reshape(Cout, Cin * 9), bias.reshape(Cout, 1)], axis=1)

    return pl.pallas_call(
        _conv3x3_body,
        grid=(B,),
        in_specs=[
            pl.BlockSpec(memory_space=pltpu.SMEM),
            pl.BlockSpec((1, Cin, H, W), lambda i: (i, 0, 0, 0)),
        ],
        out_specs=pl.BlockSpec((1, Cout, H, W), lambda i: (i, 0, 0, 0)),
        out_shape=jax.ShapeDtypeStruct((B, Cout, H, W), jnp.float32),
        compiler_params=pltpu.CompilerParams(
            dimension_semantics=("parallel",)),
    )(w2, x)
